# baseline (speedup 1.0000x reference)
import functools

import jax
import jax.numpy as jnp
from jax import lax
from jax.experimental import pallas as pl
from jax.experimental.pallas import tpu as pltpu
from jax.experimental.pallas import tpu_sc as plsc

NC = 2
NS = 16


_SQRT_HALF = 0.7071067811865476


def _gelu(x):
    return 0.5 * x * (1.0 + lax.erf(x * _SQRT_HALF))


def _sc_mesh():
    return plsc.VectorSubcoreMesh(
        core_axis_name="core", subcore_axis_name="subcore",
        num_cores=NC, num_subcores=NS)


def _gather(faug, idx, Ep, FW):
    out_t = jax.ShapeDtypeStruct((Ep, FW), jnp.float32)

    @functools.partial(pl.kernel, out_type=out_t, mesh=_sc_mesh())
    def gk(faug_hbm, idx_hbm, o_hbm):
        def body(idx_v, o_v):
            pltpu.sync_copy(faug_hbm.at[idx_v.at[0]], o_v)

        GW = 128
        pltpu.emit_pipeline(
            body,
            grid=(Ep // GW,),
            in_specs=[pl.BlockSpec((1, GW), lambda i: (0, i))],
            out_specs=[pl.BlockSpec((GW, FW), lambda i: (i, 0))],
            core_axis_name=("core", "subcore"),
            dimension_semantics=(pltpu.PARALLEL,),
        )(idx_hbm, o_hbm)

    return gk(faug, idx)


def _mlp_body(s_ref, t_ref, w1s_ref, w1t_ref, b1_ref, w2_ref, b2_ref,
              w3_ref, b3_ref, h_ref):
    f32 = jnp.float32
    bf16 = jnp.bfloat16
    a = (jnp.dot(s_ref[...], w1s_ref[...], preferred_element_type=f32)
         + jnp.dot(t_ref[...], w1t_ref[...], preferred_element_type=f32)
         + b1_ref[...])
    h = _gelu(a).astype(bf16)
    h = _gelu(jnp.dot(h, w2_ref[...], preferred_element_type=f32)
              + b2_ref[...]).astype(bf16)
    h_ref[...] = _gelu(jnp.dot(h, w3_ref[...], preferred_element_type=f32)
                       + b3_ref[...])


def _edge_mlp(ST, W1s, W1t, b1, W2, b2, W3, b3, Ep, FW, BK=512):
    H = W1s.shape[1]
    nblk = Ep // BK
    return pl.pallas_call(
        _mlp_body,
        grid=(nblk,),
        in_specs=[
            pl.BlockSpec((BK, FW), lambda i: (i, 0)),
            pl.BlockSpec((BK, FW), lambda i: (i + nblk, 0)),
            pl.BlockSpec((FW, H), lambda i: (0, 0)),
            pl.BlockSpec((FW, H), lambda i: (0, 0)),
            pl.BlockSpec((1, H), lambda i: (0, 0)),
            pl.BlockSpec((H, H), lambda i: (0, 0)),
            pl.BlockSpec((1, H), lambda i: (0, 0)),
            pl.BlockSpec((H, H), lambda i: (0, 0)),
            pl.BlockSpec((1, H), lambda i: (0, 0)),
        ],
        out_specs=pl.BlockSpec((BK, H), lambda i: (i, 0)),
        out_shape=jax.ShapeDtypeStruct((Ep, H), jnp.float32),
    )(ST, ST, W1s, W1t, b1.reshape(1, H), W2, b2.reshape(1, H),
      W3, b3.reshape(1, H))


def _segsum(h, di, zeros_np, ones128, Ep, Np, H):
    ROWS = Np // NS
    EPW = Ep // NS
    NB = EPW // 128
    f32 = jnp.float32
    out_t = [jax.ShapeDtypeStruct((Np, H), f32),
             jax.ShapeDtypeStruct((Np, 128), f32)]

    @functools.partial(
        pl.kernel, out_type=out_t, mesh=_sc_mesh(),
        scratch_types=[
            pltpu.VMEM_SHARED((Np, 128), f32),
            pltpu.VMEM((128, 128), f32),
            pltpu.VMEM((1, 128), jnp.int32),
            pltpu.VMEM((128, 128), f32),
        ])
    def sk(h_hbm, di_hbm, zeros_hbm, ones_hbm, sums_hbm, cnt_hbm,
           acc_sh, buf_v, idx_v, ones_v):
        c = lax.axis_index("core")
        s = lax.axis_index("subcore")
        rows0 = s * ROWS
        e0 = s * EPW
        for kk in range(4 // NC):
            col = (c * (4 // NC) + kk) * 128
            pltpu.sync_copy(zeros_hbm.at[pl.ds(rows0, ROWS), :],
                            acc_sh.at[pl.ds(rows0, ROWS), :])
            plsc.subcore_barrier()

            @pl.loop(0, NB)
            def _(b):
                e = e0 + b * 128
                pltpu.sync_copy(di_hbm.at[:, pl.ds(e, 128)], idx_v)
                pltpu.sync_copy(h_hbm.at[pl.ds(e, 128), pl.ds(col, 128)],
                                buf_v)
                pltpu.sync_copy(buf_v, acc_sh.at[idx_v.at[0]], add=True)

            plsc.subcore_barrier()
            pltpu.sync_copy(acc_sh.at[pl.ds(rows0, ROWS), :],
                            sums_hbm.at[pl.ds(rows0, ROWS), pl.ds(col, 128)])

        @pl.when(c == 1)
        def _():
            pltpu.sync_copy(ones_hbm, ones_v)
            pltpu.sync_copy(zeros_hbm.at[pl.ds(rows0, ROWS), :],
                            acc_sh.at[pl.ds(rows0, ROWS), :])
            plsc.subcore_barrier()

            @pl.loop(0, NB)
            def _(b):
                e = e0 + b * 128
                pltpu.sync_copy(di_hbm.at[:, pl.ds(e, 128)], idx_v)
                pltpu.sync_copy(ones_v, acc_sh.at[idx_v.at[0]], add=True)

            plsc.subcore_barrier()
            pltpu.sync_copy(acc_sh.at[pl.ds(rows0, ROWS), :],
                            cnt_hbm.at[pl.ds(rows0, ROWS), :])

    return sk(h, di, zeros_np, ones128)


def _out_body(sum_ref, cnt_ref, wo1_ref, bo1_ref, wo2_ref, bo2_ref, o_ref):
    f32 = jnp.float32
    bf16 = jnp.bfloat16
    cnt = cnt_ref[...][:, 0:1]
    agg = jnp.where(cnt > 0, sum_ref[...] / jnp.maximum(cnt, 1.0),
                    0.0).astype(bf16)
    o = _gelu(jnp.dot(agg, wo1_ref[...], preferred_element_type=f32)
              + bo1_ref[...]).astype(bf16)
    o_ref[...] = _gelu(jnp.dot(o, wo2_ref[...], preferred_element_type=f32)
                       + bo2_ref[...])


def _out_mlp(sums, cnt, Wo1, bo1, Wo2, bo2, N):
    BN = N if N <= 1024 else 1000
    H = Wo1.shape[0]
    O = Wo2.shape[1]
    return pl.pallas_call(
        _out_body,
        grid=(N // BN,),
        in_specs=[
            pl.BlockSpec((BN, H), lambda i: (i, 0)),
            pl.BlockSpec((BN, 128), lambda i: (i, 0)),
            pl.BlockSpec((H, H), lambda i: (0, 0)),
            pl.BlockSpec((1, H), lambda i: (0, 0)),
            pl.BlockSpec((H, O), lambda i: (0, 0)),
            pl.BlockSpec((1, O), lambda i: (0, 0)),
        ],
        out_specs=pl.BlockSpec((BN, O), lambda i: (i, 0)),
        out_shape=jax.ShapeDtypeStruct((N, O), jnp.float32),
    )(sums, cnt, Wo1.astype(jnp.bfloat16), bo1.reshape(1, H),
      Wo2.astype(jnp.bfloat16), bo2.reshape(1, O))


def kernel(features, points, l0_edges, We1, be1, We2, be2, We3, be3,
           Wo1, bo1, Wo2, bo2):
    N, D = features.shape
    E = l0_edges.shape[0]
    H = We2.shape[0]
    WB = 2 * D
    PW = WB // 2
    Ep = -(-E // 4096) * 4096
    Np = -(-(N + 48) // (NS * 8)) * (NS * 8)

    f32 = jnp.float32
    bf16 = jnp.bfloat16
    faug = jnp.concatenate(
        [features, points, jnp.zeros((N, WB - D - 3), f32)],
        axis=1).astype(bf16)
    packed = lax.bitcast_convert_type(faug.reshape(N, PW, 2), f32)

    edges = l0_edges.astype(jnp.int32)
    pad = Ep - E
    src = jnp.concatenate([edges[:, 0], jnp.zeros((pad,), jnp.int32)])
    dst = jnp.concatenate([edges[:, 1], jnp.zeros((pad,), jnp.int32)])
    dseg = jnp.concatenate(
        [edges[:, 1], N + (jnp.arange(pad, dtype=jnp.int32) % 48)])
    gidx = jnp.concatenate([src, dst]).reshape(1, 2 * Ep)
    di = dseg.reshape(1, Ep)

    Wc = jnp.concatenate(
        [We1[2 * D:], jnp.zeros((WB - D - 3, H), f32)], axis=0)
    W1s = jnp.concatenate([We1[:D], Wc], axis=0).astype(bf16)
    W1t = jnp.concatenate([We1[D:2 * D], -Wc], axis=0).astype(bf16)

    STp = _gather(packed, gidx, 2 * Ep, PW)
    STb = lax.bitcast_convert_type(STp, bf16).reshape(2 * Ep, WB)
    h = _edge_mlp(STb, W1s, W1t, be1, We2.astype(bf16), be2,
                  We3.astype(bf16), be3, Ep, WB)

    zeros_np = jnp.zeros((Np, 128), f32)
    ones128 = jnp.ones((128, 128), f32)
    sums, cnt = _segsum(h, di, zeros_np, ones128, Ep, Np, H)

    return _out_mlp(sums, cnt, Wo1, bo1, Wo2, bo2, N)


# speedup vs baseline: 2.6769x; 2.6769x over previous
import functools

import jax
import jax.numpy as jnp
from jax import lax
from jax.experimental import pallas as pl
from jax.experimental.pallas import tpu as pltpu
from jax.experimental.pallas import tpu_sc as plsc

NC = 2
NS = 16


_SQRT_HALF = 0.7071067811865476


def _gelu(x):
    return 0.5 * x * (1.0 + lax.erf(x * _SQRT_HALF))


def _sc_mesh():
    return plsc.VectorSubcoreMesh(
        core_axis_name="core", subcore_axis_name="subcore",
        num_cores=NC, num_subcores=NS)


def _gather(faug, idx, Ep, FW):
    out_t = jax.ShapeDtypeStruct((Ep, FW), jnp.float32)

    @functools.partial(pl.kernel, out_type=out_t, mesh=_sc_mesh())
    def gk(faug_hbm, idx_hbm, o_hbm):
        def body(idx_v, o_v):
            pltpu.sync_copy(faug_hbm.at[idx_v.at[0]], o_v)

        GW = 128
        pltpu.emit_pipeline(
            body,
            grid=(Ep // GW,),
            in_specs=[pl.BlockSpec((1, GW), lambda i: (0, i))],
            out_specs=[pl.BlockSpec((GW, FW), lambda i: (i, 0))],
            core_axis_name=("core", "subcore"),
            dimension_semantics=(pltpu.PARALLEL,),
        )(idx_hbm, o_hbm)

    return gk(faug, idx)


def _unpack_pair(x_f32):
    w = lax.bitcast_convert_type(x_f32, jnp.int32)
    a = lax.bitcast_convert_type(w << 16, jnp.float32)
    b = lax.bitcast_convert_type(w & jnp.int32(-65536), jnp.float32)
    return jnp.concatenate([a, b], axis=1).astype(jnp.bfloat16)


def _mlp_body(s_ref, t_ref, w1s_ref, w1t_ref, b1_ref, w2_ref, b2_ref,
              w3_ref, b3_ref, h_ref):
    f32 = jnp.float32
    bf16 = jnp.bfloat16
    s = _unpack_pair(s_ref[...])
    t = _unpack_pair(t_ref[...])
    a = (jnp.dot(s, w1s_ref[...], preferred_element_type=f32)
         + jnp.dot(t, w1t_ref[...], preferred_element_type=f32)
         + b1_ref[...])
    h = _gelu(a).astype(bf16)
    h = _gelu(jnp.dot(h, w2_ref[...], preferred_element_type=f32)
              + b2_ref[...]).astype(bf16)
    h_ref[...] = _gelu(jnp.dot(h, w3_ref[...], preferred_element_type=f32)
                       + b3_ref[...])


def _edge_mlp(ST, W1s, W1t, b1, W2, b2, W3, b3, Ep, PW, BK=512):
    H = W1s.shape[1]
    WB = W1s.shape[0]
    nblk = Ep // BK
    return pl.pallas_call(
        _mlp_body,
        grid=(nblk,),
        in_specs=[
            pl.BlockSpec((BK, PW), lambda i: (i, 0)),
            pl.BlockSpec((BK, PW), lambda i: (i + nblk, 0)),
            pl.BlockSpec((WB, H), lambda i: (0, 0)),
            pl.BlockSpec((WB, H), lambda i: (0, 0)),
            pl.BlockSpec((1, H), lambda i: (0, 0)),
            pl.BlockSpec((H, H), lambda i: (0, 0)),
            pl.BlockSpec((1, H), lambda i: (0, 0)),
            pl.BlockSpec((H, H), lambda i: (0, 0)),
            pl.BlockSpec((1, H), lambda i: (0, 0)),
        ],
        out_specs=pl.BlockSpec((BK, H), lambda i: (i, 0)),
        out_shape=jax.ShapeDtypeStruct((Ep, H), jnp.float32),
    )(ST, ST, W1s, W1t, b1.reshape(1, H), W2, b2.reshape(1, H),
      W3, b3.reshape(1, H))


def _segsum(h, di, zeros_np, ones128, Ep, Np, H):
    ROWS = Np // NS
    EPW = Ep // NS
    NB = EPW // 128
    f32 = jnp.float32
    out_t = [jax.ShapeDtypeStruct((Np, H), f32),
             jax.ShapeDtypeStruct((Np, 128), f32)]

    @functools.partial(
        pl.kernel, out_type=out_t, mesh=_sc_mesh(),
        scratch_types=[
            pltpu.VMEM_SHARED((Np, 128), f32),
            pltpu.VMEM((128, 128), f32),
            pltpu.VMEM((1, 128), jnp.int32),
            pltpu.VMEM((128, 128), f32),
        ])
    def sk(h_hbm, di_hbm, zeros_hbm, ones_hbm, sums_hbm, cnt_hbm,
           acc_sh, buf_v, idx_v, ones_v):
        c = lax.axis_index("core")
        s = lax.axis_index("subcore")
        rows0 = s * ROWS
        e0 = s * EPW
        for kk in range(4 // NC):
            col = (c * (4 // NC) + kk) * 128
            pltpu.sync_copy(zeros_hbm.at[pl.ds(rows0, ROWS), :],
                            acc_sh.at[pl.ds(rows0, ROWS), :])
            plsc.subcore_barrier()

            @pl.loop(0, NB)
            def _(b):
                e = e0 + b * 128
                pltpu.sync_copy(di_hbm.at[:, pl.ds(e, 128)], idx_v)
                pltpu.sync_copy(h_hbm.at[pl.ds(e, 128), pl.ds(col, 128)],
                                buf_v)
                pltpu.sync_copy(buf_v, acc_sh.at[idx_v.at[0]], add=True)

            plsc.subcore_barrier()
            pltpu.sync_copy(acc_sh.at[pl.ds(rows0, ROWS), :],
                            sums_hbm.at[pl.ds(rows0, ROWS), pl.ds(col, 128)])

        @pl.when(c == 1)
        def _():
            pltpu.sync_copy(ones_hbm, ones_v)
            pltpu.sync_copy(zeros_hbm.at[pl.ds(rows0, ROWS), :],
                            acc_sh.at[pl.ds(rows0, ROWS), :])
            plsc.subcore_barrier()

            @pl.loop(0, NB)
            def _(b):
                e = e0 + b * 128
                pltpu.sync_copy(di_hbm.at[:, pl.ds(e, 128)], idx_v)
                pltpu.sync_copy(ones_v, acc_sh.at[idx_v.at[0]], add=True)

            plsc.subcore_barrier()
            pltpu.sync_copy(acc_sh.at[pl.ds(rows0, ROWS), :],
                            cnt_hbm.at[pl.ds(rows0, ROWS), :])

    return sk(h, di, zeros_np, ones128)


def _out_body(sum_ref, cnt_ref, wo1_ref, bo1_ref, wo2_ref, bo2_ref, o_ref):
    f32 = jnp.float32
    bf16 = jnp.bfloat16
    cnt = cnt_ref[...][:, 0:1]
    agg = jnp.where(cnt > 0, sum_ref[...] / jnp.maximum(cnt, 1.0),
                    0.0).astype(bf16)
    o = _gelu(jnp.dot(agg, wo1_ref[...], preferred_element_type=f32)
              + bo1_ref[...]).astype(bf16)
    o_ref[...] = _gelu(jnp.dot(o, wo2_ref[...], preferred_element_type=f32)
                       + bo2_ref[...])


def _out_mlp(sums, cnt, Wo1, bo1, Wo2, bo2, N):
    BN = N if N <= 1024 else 1000
    H = Wo1.shape[0]
    O = Wo2.shape[1]
    return pl.pallas_call(
        _out_body,
        grid=(N // BN,),
        in_specs=[
            pl.BlockSpec((BN, H), lambda i: (i, 0)),
            pl.BlockSpec((BN, 128), lambda i: (i, 0)),
            pl.BlockSpec((H, H), lambda i: (0, 0)),
            pl.BlockSpec((1, H), lambda i: (0, 0)),
            pl.BlockSpec((H, O), lambda i: (0, 0)),
            pl.BlockSpec((1, O), lambda i: (0, 0)),
        ],
        out_specs=pl.BlockSpec((BN, O), lambda i: (i, 0)),
        out_shape=jax.ShapeDtypeStruct((N, O), jnp.float32),
    )(sums, cnt, Wo1.astype(jnp.bfloat16), bo1.reshape(1, H),
      Wo2.astype(jnp.bfloat16), bo2.reshape(1, O))


def kernel(features, points, l0_edges, We1, be1, We2, be2, We3, be3,
           Wo1, bo1, Wo2, bo2):
    N, D = features.shape
    E = l0_edges.shape[0]
    H = We2.shape[0]
    WB = 2 * D
    PW = WB // 2
    Ep = -(-E // 4096) * 4096
    Np = -(-(N + 48) // (NS * 8)) * (NS * 8)

    f32 = jnp.float32
    bf16 = jnp.bfloat16
    faug = jnp.concatenate(
        [features, points, jnp.zeros((N, WB - D - 3), f32)],
        axis=1).astype(bf16)
    au = lax.bitcast_convert_type(faug[:, :PW], jnp.uint16).astype(jnp.uint32)
    bu = lax.bitcast_convert_type(faug[:, PW:], jnp.uint16).astype(jnp.uint32)
    packed = lax.bitcast_convert_type(au | (bu << 16), f32)

    edges = l0_edges.astype(jnp.int32)
    pad = Ep - E
    src = jnp.concatenate([edges[:, 0], jnp.zeros((pad,), jnp.int32)])
    dst = jnp.concatenate([edges[:, 1], jnp.zeros((pad,), jnp.int32)])
    dseg = jnp.concatenate(
        [edges[:, 1], N + (jnp.arange(pad, dtype=jnp.int32) % 48)])
    gidx = jnp.concatenate([src, dst]).reshape(1, 2 * Ep)
    di = dseg.reshape(1, Ep)

    Wc = jnp.concatenate(
        [We1[2 * D:], jnp.zeros((WB - D - 3, H), f32)], axis=0)
    W1s = jnp.concatenate([We1[:D], Wc], axis=0).astype(bf16)
    W1t = jnp.concatenate([We1[D:2 * D], -Wc], axis=0).astype(bf16)

    STp = _gather(packed, gidx, 2 * Ep, PW)
    h = _edge_mlp(STp, W1s, W1t, be1, We2.astype(bf16), be2,
                  We3.astype(bf16), be3, Ep, PW)

    zeros_np = jnp.zeros((Np, 128), f32)
    ones128 = jnp.ones((128, 128), f32)
    sums, cnt = _segsum(h, di, zeros_np, ones128, Ep, Np, H)

    return _out_mlp(sums, cnt, Wo1, bo1, Wo2, bo2, N)


# speedup vs baseline: 3.2697x; 1.2214x over previous
import functools

import jax
import jax.numpy as jnp
from jax import lax
from jax.experimental import pallas as pl
from jax.experimental.pallas import tpu as pltpu
from jax.experimental.pallas import tpu_sc as plsc

NC = 2
NS = 16


_SQRT_HALF = 0.7071067811865476


def _gelu(x):
    return 0.5 * x * (1.0 + lax.erf(x * _SQRT_HALF))


def _sc_mesh():
    return plsc.VectorSubcoreMesh(
        core_axis_name="core", subcore_axis_name="subcore",
        num_cores=NC, num_subcores=NS)


def _gather(faug, idx, Ep, FW):
    out_t = jax.ShapeDtypeStruct((Ep, FW), jnp.float32)

    @functools.partial(pl.kernel, out_type=out_t, mesh=_sc_mesh())
    def gk(faug_hbm, idx_hbm, o_hbm):
        def body(idx_v, o_v):
            pltpu.sync_copy(faug_hbm.at[idx_v.at[0]], o_v)

        GW = 128
        pltpu.emit_pipeline(
            body,
            grid=(Ep // GW,),
            in_specs=[pl.BlockSpec((1, GW), lambda i: (0, i))],
            out_specs=[pl.BlockSpec((GW, FW), lambda i: (i, 0))],
            core_axis_name=("core", "subcore"),
            dimension_semantics=(pltpu.PARALLEL,),
        )(idx_hbm, o_hbm)

    return gk(faug, idx)


def _unpack_pair(x_f32):
    w = lax.bitcast_convert_type(x_f32, jnp.int32)
    a = lax.bitcast_convert_type(w << 16, jnp.float32)
    b = lax.bitcast_convert_type(w & jnp.int32(-65536), jnp.float32)
    return jnp.concatenate([a, b], axis=1).astype(jnp.bfloat16)


def _mlp_body(s_ref, t_ref, w1s_ref, w1t_ref, b1_ref, w2_ref, b2_ref,
              w3_ref, b3_ref, h_ref):
    f32 = jnp.float32
    bf16 = jnp.bfloat16
    s = _unpack_pair(s_ref[...])
    t = _unpack_pair(t_ref[...])
    a = (jnp.dot(s, w1s_ref[...], preferred_element_type=f32)
         + jnp.dot(t, w1t_ref[...], preferred_element_type=f32)
         + b1_ref[...])
    h = _gelu(a).astype(bf16)
    h = _gelu(jnp.dot(h, w2_ref[...], preferred_element_type=f32)
              + b2_ref[...]).astype(bf16)
    h_ref[...] = _gelu(jnp.dot(h, w3_ref[...], preferred_element_type=f32)
                       + b3_ref[...])


def _edge_mlp(ST, W1s, W1t, b1, W2, b2, W3, b3, Ep, PW, BK=512):
    H = W1s.shape[1]
    WB = W1s.shape[0]
    nblk = Ep // BK
    return pl.pallas_call(
        _mlp_body,
        grid=(nblk,),
        in_specs=[
            pl.BlockSpec((BK, PW), lambda i: (i, 0)),
            pl.BlockSpec((BK, PW), lambda i: (i + nblk, 0)),
            pl.BlockSpec((WB, H), lambda i: (0, 0)),
            pl.BlockSpec((WB, H), lambda i: (0, 0)),
            pl.BlockSpec((1, H), lambda i: (0, 0)),
            pl.BlockSpec((H, H), lambda i: (0, 0)),
            pl.BlockSpec((1, H), lambda i: (0, 0)),
            pl.BlockSpec((H, H), lambda i: (0, 0)),
            pl.BlockSpec((1, H), lambda i: (0, 0)),
        ],
        out_specs=pl.BlockSpec((BK, H), lambda i: (i, 0)),
        out_shape=jax.ShapeDtypeStruct((Ep, H), jnp.float32),
    )(ST, ST, W1s, W1t, b1.reshape(1, H), W2, b2.reshape(1, H),
      W3, b3.reshape(1, H))


def _segsum(h, di, zeros_np, Ep, Np, H):
    ROWS = Np // NS
    EPW = Ep // NS
    NB = EPW // 128
    f32 = jnp.float32
    out_t = jax.ShapeDtypeStruct((Np, H), f32)

    @functools.partial(
        pl.kernel, out_type=out_t, mesh=_sc_mesh(),
        scratch_types=[
            pltpu.VMEM_SHARED((Np, 128), f32),
            pltpu.VMEM((128, 128), f32),
            pltpu.VMEM((1, 128), jnp.int32),
        ])
    def sk(h_hbm, di_hbm, zeros_hbm, sums_hbm, acc_sh, buf_v, idx_v):
        c = lax.axis_index("core")
        s = lax.axis_index("subcore")
        rows0 = s * ROWS
        e0 = s * EPW
        for kk in range(4 // NC):
            col = (c * (4 // NC) + kk) * 128
            pltpu.sync_copy(zeros_hbm.at[pl.ds(rows0, ROWS), :],
                            acc_sh.at[pl.ds(rows0, ROWS), :])
            plsc.subcore_barrier()

            @pl.loop(0, NB)
            def _(b):
                e = e0 + b * 128
                pltpu.sync_copy(di_hbm.at[:, pl.ds(e, 128)], idx_v)
                pltpu.sync_copy(h_hbm.at[pl.ds(e, 128), pl.ds(col, 128)],
                                buf_v)
                pltpu.sync_copy(buf_v, acc_sh.at[idx_v.at[0]], add=True)

            plsc.subcore_barrier()
            pltpu.sync_copy(acc_sh.at[pl.ds(rows0, ROWS), :],
                            sums_hbm.at[pl.ds(rows0, ROWS), pl.ds(col, 128)])

    return sk(h, di, zeros_np)


def _counts(di, zeros_cnt, ones8, Ep, Np):
    ROWS = Np // NS
    EPW = Ep // (NS * NC)
    NB = EPW // 128
    f32 = jnp.float32
    out_t = jax.ShapeDtypeStruct((NC, Np, 128), f32)

    @functools.partial(
        pl.kernel, out_type=out_t, mesh=_sc_mesh(),
        scratch_types=[
            pltpu.VMEM_SHARED((Np, 128), f32),
            pltpu.VMEM((1, 128), jnp.int32),
            pltpu.VMEM((128, 128), f32),
        ])
    def ck(di_hbm, zeros_hbm, ones_hbm, cnt_hbm, acc_sh, idx_v, ones_v):
        c = lax.axis_index("core")
        s = lax.axis_index("subcore")
        rows0 = s * ROWS
        e0 = (c * NS + s) * EPW
        pltpu.sync_copy(ones_hbm, ones_v)
        pltpu.sync_copy(zeros_hbm.at[pl.ds(rows0, ROWS), :],
                        acc_sh.at[pl.ds(rows0, ROWS), :])
        plsc.subcore_barrier()

        @pl.loop(0, NB)
        def _(b):
            e = e0 + b * 128
            pltpu.sync_copy(di_hbm.at[:, pl.ds(e, 128)], idx_v)
            pltpu.sync_copy(ones_v, acc_sh.at[idx_v.at[0]], add=True)

        plsc.subcore_barrier()
        pltpu.sync_copy(acc_sh.at[pl.ds(rows0, ROWS), :],
                        cnt_hbm.at[c, pl.ds(rows0, ROWS), :])

    return ck(di, zeros_cnt, ones8)


def _out_body(s0_ref, s1_ref, s2_ref, s3_ref, c0_ref, c1_ref,
              wo1_ref, bo1_ref, wo2_ref, bo2_ref, o_ref):
    f32 = jnp.float32
    bf16 = jnp.bfloat16
    ssum = ((s0_ref[...] + s1_ref[...]) + (s2_ref[...] + s3_ref[...]))
    cnt = (c0_ref[0] + c1_ref[0])[:, 0:1]
    agg = jnp.where(cnt > 0, ssum / jnp.maximum(cnt, 1.0),
                    0.0).astype(bf16)
    o = _gelu(jnp.dot(agg, wo1_ref[...], preferred_element_type=f32)
              + bo1_ref[...]).astype(bf16)
    o_ref[...] = _gelu(jnp.dot(o, wo2_ref[...], preferred_element_type=f32)
                       + bo2_ref[...])


def _out_mlp(sums_list, cnt, Wo1, bo1, Wo2, bo2, N):
    BN = N if N <= 1024 else 1000
    H = Wo1.shape[0]
    O = Wo2.shape[1]
    return pl.pallas_call(
        _out_body,
        grid=(N // BN,),
        in_specs=[
            pl.BlockSpec((BN, H), lambda i: (i, 0)),
            pl.BlockSpec((BN, H), lambda i: (i, 0)),
            pl.BlockSpec((BN, H), lambda i: (i, 0)),
            pl.BlockSpec((BN, H), lambda i: (i, 0)),
            pl.BlockSpec((1, BN, 128), lambda i: (0, i, 0)),
            pl.BlockSpec((1, BN, 128), lambda i: (1, i, 0)),
            pl.BlockSpec((H, H), lambda i: (0, 0)),
            pl.BlockSpec((1, H), lambda i: (0, 0)),
            pl.BlockSpec((H, O), lambda i: (0, 0)),
            pl.BlockSpec((1, O), lambda i: (0, 0)),
        ],
        out_specs=pl.BlockSpec((BN, O), lambda i: (i, 0)),
        out_shape=jax.ShapeDtypeStruct((N, O), jnp.float32),
    )(*sums_list, cnt, cnt, Wo1.astype(jnp.bfloat16), bo1.reshape(1, H),
      Wo2.astype(jnp.bfloat16), bo2.reshape(1, O))


def kernel(features, points, l0_edges, We1, be1, We2, be2, We3, be3,
           Wo1, bo1, Wo2, bo2):
    N, D = features.shape
    E = l0_edges.shape[0]
    H = We2.shape[0]
    WB = 2 * D
    PW = WB // 2
    Ep = -(-E // 4096) * 4096
    Np = -(-(N + 48) // (NS * 8)) * (NS * 8)

    f32 = jnp.float32
    bf16 = jnp.bfloat16
    faug = jnp.concatenate(
        [features, points, jnp.zeros((N, WB - D - 3), f32)],
        axis=1).astype(bf16)
    au = lax.bitcast_convert_type(faug[:, :PW], jnp.uint16).astype(jnp.uint32)
    bu = lax.bitcast_convert_type(faug[:, PW:], jnp.uint16).astype(jnp.uint32)
    packed = lax.bitcast_convert_type(au | (bu << 16), f32)

    edges = l0_edges.astype(jnp.int32)
    pad = Ep - E
    src = jnp.concatenate([edges[:, 0], jnp.zeros((pad,), jnp.int32)])
    dst = jnp.concatenate([edges[:, 1], jnp.zeros((pad,), jnp.int32)])
    dseg = jnp.concatenate(
        [edges[:, 1], N + (jnp.arange(pad, dtype=jnp.int32) % 48)])
    NCH = 4
    Ec = Ep // NCH
    src4 = src.reshape(NCH, Ec)
    dst4 = dst.reshape(NCH, Ec)
    dseg4 = dseg.reshape(NCH, Ec)
    di = dseg.reshape(1, Ep)

    Wc = jnp.concatenate(
        [We1[2 * D:], jnp.zeros((WB - D - 3, H), f32)], axis=0)
    W1s = jnp.concatenate([We1[:D], Wc], axis=0).astype(bf16)
    W1t = jnp.concatenate([We1[D:2 * D], -Wc], axis=0).astype(bf16)

    zeros_np = jnp.zeros((Np, 128), f32)
    ones128 = jnp.ones((128, 128), f32)
    We2b = We2.astype(bf16)
    We3b = We3.astype(bf16)

    cnt = _counts(di, zeros_np, ones128, Ep, Np)
    sums_list = []
    for q in range(NCH):
        gq = jnp.concatenate([src4[q], dst4[q]]).reshape(1, 2 * Ec)
        STq = _gather(packed, gq, 2 * Ec, PW)
        hq = _edge_mlp(STq, W1s, W1t, be1, We2b, be2, We3b, be3, Ec, PW)
        sums_list.append(
            _segsum(hq, dseg4[q].reshape(1, Ec), zeros_np, Ec, Np, H))

    return _out_mlp(sums_list, cnt, Wo1, bo1, Wo2, bo2, N)


# speedup vs baseline: 3.2841x; 1.0044x over previous
import functools

import jax
import jax.numpy as jnp
from jax import lax
from jax.experimental import pallas as pl
from jax.experimental.pallas import tpu as pltpu
from jax.experimental.pallas import tpu_sc as plsc

NC = 2
NS = 16


_SQRT_HALF = 0.7071067811865476


def _gelu(x):
    return 0.5 * x * (1.0 + lax.erf(x * _SQRT_HALF))


def _sc_mesh():
    return plsc.VectorSubcoreMesh(
        core_axis_name="core", subcore_axis_name="subcore",
        num_cores=NC, num_subcores=NS)


def _gather(faug, idx, Ep, FW):
    out_t = jax.ShapeDtypeStruct((Ep, FW), jnp.float32)

    @functools.partial(pl.kernel, out_type=out_t, mesh=_sc_mesh())
    def gk(faug_hbm, idx_hbm, o_hbm):
        def body(idx_v, o_v):
            def two_streams(s1, s2):
                c0 = pltpu.async_copy(
                    faug_hbm.at[idx_v.at[0, pl.ds(0, 64)]],
                    o_v.at[pl.ds(0, 64)], s1)
                c1 = pltpu.async_copy(
                    faug_hbm.at[idx_v.at[0, pl.ds(64, 64)]],
                    o_v.at[pl.ds(64, 64)], s2)
                c0.wait()
                c1.wait()

            pl.run_scoped(two_streams, pltpu.SemaphoreType.DMA,
                          pltpu.SemaphoreType.DMA)

        GW = 128
        pltpu.emit_pipeline(
            body,
            grid=(Ep // GW,),
            in_specs=[pl.BlockSpec((1, GW), lambda i: (0, i))],
            out_specs=[pl.BlockSpec((GW, FW), lambda i: (i, 0))],
            core_axis_name=("core", "subcore"),
            dimension_semantics=(pltpu.PARALLEL,),
        )(idx_hbm, o_hbm)

    return gk(faug, idx)


def _unpack_pair(x_f32):
    w = lax.bitcast_convert_type(x_f32, jnp.int32)
    a = lax.bitcast_convert_type(w << 16, jnp.float32)
    b = lax.bitcast_convert_type(w & jnp.int32(-65536), jnp.float32)
    return jnp.concatenate([a, b], axis=1).astype(jnp.bfloat16)


def _mlp_body(s_ref, t_ref, w1s_ref, w1t_ref, b1_ref, w2_ref, b2_ref,
              w3_ref, b3_ref, h_ref):
    f32 = jnp.float32
    bf16 = jnp.bfloat16
    s = _unpack_pair(s_ref[...])
    t = _unpack_pair(t_ref[...])
    a = (jnp.dot(s, w1s_ref[...], preferred_element_type=f32)
         + jnp.dot(t, w1t_ref[...], preferred_element_type=f32)
         + b1_ref[...])
    h = _gelu(a).astype(bf16)
    h = _gelu(jnp.dot(h, w2_ref[...], preferred_element_type=f32)
              + b2_ref[...]).astype(bf16)
    h_ref[...] = _gelu(jnp.dot(h, w3_ref[...], preferred_element_type=f32)
                       + b3_ref[...])


def _edge_mlp(ST, W1s, W1t, b1, W2, b2, W3, b3, Ep, PW, BK=512):
    H = W1s.shape[1]
    WB = W1s.shape[0]
    nblk = Ep // BK
    return pl.pallas_call(
        _mlp_body,
        grid=(nblk,),
        in_specs=[
            pl.BlockSpec((BK, PW), lambda i: (i, 0)),
            pl.BlockSpec((BK, PW), lambda i: (i + nblk, 0)),
            pl.BlockSpec((WB, H), lambda i: (0, 0)),
            pl.BlockSpec((WB, H), lambda i: (0, 0)),
            pl.BlockSpec((1, H), lambda i: (0, 0)),
            pl.BlockSpec((H, H), lambda i: (0, 0)),
            pl.BlockSpec((1, H), lambda i: (0, 0)),
            pl.BlockSpec((H, H), lambda i: (0, 0)),
            pl.BlockSpec((1, H), lambda i: (0, 0)),
        ],
        out_specs=pl.BlockSpec((BK, H), lambda i: (i, 0)),
        out_shape=jax.ShapeDtypeStruct((Ep, H), jnp.float32),
    )(ST, ST, W1s, W1t, b1.reshape(1, H), W2, b2.reshape(1, H),
      W3, b3.reshape(1, H))


def _segsum(h, di, zeros_np, Ep, Np, H):
    ROWS = Np // NS
    EPW = Ep // NS
    NB = EPW // 128
    f32 = jnp.float32
    out_t = jax.ShapeDtypeStruct((Np, H), f32)

    @functools.partial(
        pl.kernel, out_type=out_t, mesh=_sc_mesh(),
        scratch_types=[
            pltpu.VMEM_SHARED((Np, 128), f32),
            pltpu.VMEM((128, 128), f32),
            pltpu.VMEM((1, 128), jnp.int32),
        ])
    def sk(h_hbm, di_hbm, zeros_hbm, sums_hbm, acc_sh, buf_v, idx_v):
        c = lax.axis_index("core")
        s = lax.axis_index("subcore")
        rows0 = s * ROWS
        e0 = s * EPW
        for kk in range(4 // NC):
            col = (c * (4 // NC) + kk) * 128
            pltpu.sync_copy(zeros_hbm.at[pl.ds(rows0, ROWS), :],
                            acc_sh.at[pl.ds(rows0, ROWS), :])
            plsc.subcore_barrier()

            @pl.loop(0, NB)
            def _(b):
                e = e0 + b * 128
                pltpu.sync_copy(di_hbm.at[:, pl.ds(e, 128)], idx_v)
                pltpu.sync_copy(h_hbm.at[pl.ds(e, 128), pl.ds(col, 128)],
                                buf_v)
                pltpu.sync_copy(buf_v, acc_sh.at[idx_v.at[0]], add=True)

            plsc.subcore_barrier()
            pltpu.sync_copy(acc_sh.at[pl.ds(rows0, ROWS), :],
                            sums_hbm.at[pl.ds(rows0, ROWS), pl.ds(col, 128)])

    return sk(h, di, zeros_np)


def _counts(di, zeros_cnt, ones8, Ep, Np):
    ROWS = Np // NS
    EPW = Ep // (NS * NC)
    NB = EPW // 128
    f32 = jnp.float32
    out_t = jax.ShapeDtypeStruct((NC, Np, 128), f32)

    @functools.partial(
        pl.kernel, out_type=out_t, mesh=_sc_mesh(),
        scratch_types=[
            pltpu.VMEM_SHARED((Np, 128), f32),
            pltpu.VMEM((1, 128), jnp.int32),
            pltpu.VMEM((128, 128), f32),
        ])
    def ck(di_hbm, zeros_hbm, ones_hbm, cnt_hbm, acc_sh, idx_v, ones_v):
        c = lax.axis_index("core")
        s = lax.axis_index("subcore")
        rows0 = s * ROWS
        e0 = (c * NS + s) * EPW
        pltpu.sync_copy(ones_hbm, ones_v)
        pltpu.sync_copy(zeros_hbm.at[pl.ds(rows0, ROWS), :],
                        acc_sh.at[pl.ds(rows0, ROWS), :])
        plsc.subcore_barrier()

        @pl.loop(0, NB)
        def _(b):
            e = e0 + b * 128
            pltpu.sync_copy(di_hbm.at[:, pl.ds(e, 128)], idx_v)
            pltpu.sync_copy(ones_v, acc_sh.at[idx_v.at[0]], add=True)

        plsc.subcore_barrier()
        pltpu.sync_copy(acc_sh.at[pl.ds(rows0, ROWS), :],
                        cnt_hbm.at[c, pl.ds(rows0, ROWS), :])

    return ck(di, zeros_cnt, ones8)


def _out_body(s0_ref, s1_ref, s2_ref, s3_ref, c0_ref, c1_ref,
              wo1_ref, bo1_ref, wo2_ref, bo2_ref, o_ref):
    f32 = jnp.float32
    bf16 = jnp.bfloat16
    ssum = ((s0_ref[...] + s1_ref[...]) + (s2_ref[...] + s3_ref[...]))
    cnt = (c0_ref[0] + c1_ref[0])[:, 0:1]
    agg = jnp.where(cnt > 0, ssum / jnp.maximum(cnt, 1.0),
                    0.0).astype(bf16)
    o = _gelu(jnp.dot(agg, wo1_ref[...], preferred_element_type=f32)
              + bo1_ref[...]).astype(bf16)
    o_ref[...] = _gelu(jnp.dot(o, wo2_ref[...], preferred_element_type=f32)
                       + bo2_ref[...])


def _out_mlp(sums_list, cnt, Wo1, bo1, Wo2, bo2, N):
    BN = N if N <= 1024 else 1000
    H = Wo1.shape[0]
    O = Wo2.shape[1]
    return pl.pallas_call(
        _out_body,
        grid=(N // BN,),
        in_specs=[
            pl.BlockSpec((BN, H), lambda i: (i, 0)),
            pl.BlockSpec((BN, H), lambda i: (i, 0)),
            pl.BlockSpec((BN, H), lambda i: (i, 0)),
            pl.BlockSpec((BN, H), lambda i: (i, 0)),
            pl.BlockSpec((1, BN, 128), lambda i: (0, i, 0)),
            pl.BlockSpec((1, BN, 128), lambda i: (1, i, 0)),
            pl.BlockSpec((H, H), lambda i: (0, 0)),
            pl.BlockSpec((1, H), lambda i: (0, 0)),
            pl.BlockSpec((H, O), lambda i: (0, 0)),
            pl.BlockSpec((1, O), lambda i: (0, 0)),
        ],
        out_specs=pl.BlockSpec((BN, O), lambda i: (i, 0)),
        out_shape=jax.ShapeDtypeStruct((N, O), jnp.float32),
    )(*sums_list, cnt, cnt, Wo1.astype(jnp.bfloat16), bo1.reshape(1, H),
      Wo2.astype(jnp.bfloat16), bo2.reshape(1, O))


def kernel(features, points, l0_edges, We1, be1, We2, be2, We3, be3,
           Wo1, bo1, Wo2, bo2):
    N, D = features.shape
    E = l0_edges.shape[0]
    H = We2.shape[0]
    WB = 2 * D
    PW = WB // 2
    Ep = -(-E // 4096) * 4096
    Np = -(-(N + 48) // (NS * 8)) * (NS * 8)

    f32 = jnp.float32
    bf16 = jnp.bfloat16
    faug = jnp.concatenate(
        [features, points, jnp.zeros((N, WB - D - 3), f32)],
        axis=1).astype(bf16)
    au = lax.bitcast_convert_type(faug[:, :PW], jnp.uint16).astype(jnp.uint32)
    bu = lax.bitcast_convert_type(faug[:, PW:], jnp.uint16).astype(jnp.uint32)
    packed = lax.bitcast_convert_type(au | (bu << 16), f32)

    edges = l0_edges.astype(jnp.int32)
    pad = Ep - E
    src = jnp.concatenate([edges[:, 0], jnp.zeros((pad,), jnp.int32)])
    dst = jnp.concatenate([edges[:, 1], jnp.zeros((pad,), jnp.int32)])
    dseg = jnp.concatenate(
        [edges[:, 1], N + (jnp.arange(pad, dtype=jnp.int32) % 48)])
    NCH = 4
    Ec = Ep // NCH
    src4 = src.reshape(NCH, Ec)
    dst4 = dst.reshape(NCH, Ec)
    dseg4 = dseg.reshape(NCH, Ec)
    di = dseg.reshape(1, Ep)

    Wc = jnp.concatenate(
        [We1[2 * D:], jnp.zeros((WB - D - 3, H), f32)], axis=0)
    W1s = jnp.concatenate([We1[:D], Wc], axis=0).astype(bf16)
    W1t = jnp.concatenate([We1[D:2 * D], -Wc], axis=0).astype(bf16)

    zeros_np = jnp.zeros((Np, 128), f32)
    ones128 = jnp.ones((128, 128), f32)
    We2b = We2.astype(bf16)
    We3b = We3.astype(bf16)

    cnt = _counts(di, zeros_np, ones128, Ep, Np)
    sums_list = []
    for q in range(NCH):
        gq = jnp.concatenate([src4[q], dst4[q]]).reshape(1, 2 * Ec)
        STq = _gather(packed, gq, 2 * Ec, PW)
        hq = _edge_mlp(STq, W1s, W1t, be1, We2b, be2, We3b, be3, Ec, PW)
        sums_list.append(
            _segsum(hq, dseg4[q].reshape(1, Ec), zeros_np, Ec, Np, H))

    return _out_mlp(sums_list, cnt, Wo1, bo1, Wo2, bo2, N)
